# baseline (speedup 1.0000x reference)
import functools

import numpy as np

import jax
import jax.numpy as jnp
from jax import lax
from jax.experimental import pallas as pl
from jax.experimental.pallas import tpu as pltpu
from jax.sharding import Mesh, PartitionSpec as P

F32 = jnp.float32
BF16 = jnp.bfloat16

_DN = (((1,), (1,)), ((), ()))


def _dott(thin, big):
    return lax.dot_general(thin, big.astype(BF16), dimension_numbers=_DN,
                           preferred_element_type=F32,
                           precision=lax.Precision.DEFAULT)


def _pass0_body(n, bm0, params_ref, x1_ref, x2_ref, we1_ref, we2_ref,
                g1a_ref, g1b_ref, g2a_ref, g2b_ref, c1_ref, c2_ref):
    i = pl.program_id(0)
    f1 = lax.dot_general(we1_ref[...], x1_ref[...],
                         dimension_numbers=(((0,), (1,)), ((), ())),
                         preferred_element_type=F32)
    f2 = lax.dot_general(we2_ref[...], x2_ref[...],
                         dimension_numbers=(((0,), (1,)), ((), ())),
                         preferred_element_type=F32)
    g1a_ref[...] = (params_ref[0, 0] * f1).astype(BF16)
    g1b_ref[...] = (params_ref[0, 1] * f1).astype(BF16)
    g2a_ref[...] = (params_ref[0, 3] * f2).astype(BF16)
    g2b_ref[...] = (params_ref[0, 4] * f2).astype(BF16)

    @pl.when(i == 0)
    def _():
        c1_ref[...] = jnp.zeros_like(c1_ref)
        c2_ref[...] = jnp.zeros_like(c2_ref)

    o = f1.shape[0]
    valid = lax.broadcasted_iota(jnp.int32, (o, bm0), 1) < (n - i * bm0)
    zero = jnp.zeros((), F32)
    c1_ref[...] += params_ref[0, 2] * jnp.sum(
        jnp.where(valid, f1, zero), axis=1, keepdims=True)
    c2_ref[...] += params_ref[0, 5] * jnp.sum(
        jnp.where(valid, f2, zero), axis=1, keepdims=True)


def _pass1_body(n, bm, bk, as1_ref, af1_ref, as2_ref, af2_ref,
                g1a_ref, g1b_ref, g2a_ref, g2b_ref, c1_ref, c2_ref,
                wm1at_ref, wm1bt_ref, bm1_ref, wm2t_ref, bm2_ref,
                l1_ref, l2_ref, comb_ref, combb_ref):
    k = pl.program_id(1)
    nk = pl.num_programs(1)
    ks = pl.ds(k * bk, bk)

    def tiles(masked):
        a1, b1 = as1_ref[...], af1_ref[...]
        a2, b2 = as2_ref[...], af2_ref[...]
        if masked:
            valid = lax.broadcasted_iota(jnp.int32, (bm, bk), 1) < (n - k * bk)
            zero = jnp.zeros((), F32)
            a1 = jnp.where(valid, a1, zero)
            b1 = jnp.where(valid, b1, zero)
            a2 = jnp.where(valid, a2, zero)
            b2 = jnp.where(valid, b2, zero)
        p1 = _dott(g1a_ref[:, ks], a1) + _dott(g1b_ref[:, ks], b1)
        p2 = _dott(g2a_ref[:, ks], a2) + _dott(g2b_ref[:, ks], b2)
        return p1, p2

    def accumulate(p1, p2):
        @pl.when(k == 0)
        def _():
            l1_ref[...] = c1_ref[...] + p1
            l2_ref[...] = c2_ref[...] + p2

        @pl.when(k != 0)
        def _():
            l1_ref[...] += p1
            l2_ref[...] += p2

    @pl.when(k != nk - 1)
    def _():
        accumulate(*tiles(masked=False))

    @pl.when(k == nk - 1)
    def _():
        accumulate(*tiles(masked=True))
        l1 = l1_ref[...]
        l2 = l2_ref[...]
        h = (jnp.dot(wm1at_ref[...], l1, preferred_element_type=F32) +
             jnp.dot(wm1bt_ref[...], l2, preferred_element_type=F32) +
             bm1_ref[...])
        comb = jnp.dot(wm2t_ref[...], h,
                       preferred_element_type=F32) + bm2_ref[...]
        comb_ref[...] = comb
        combb_ref[...] = comb.astype(BF16)


def _pass2_body(n, bm, bk, as1_ref, as2_ref, comb_ref, wd1t_ref, wd2t_ref,
                r1_ref, r2_ref, acc1_ref, acc2_ref):
    k = pl.program_id(1)
    nk = pl.num_programs(1)
    ks = pl.ds(k * bk, bk)

    def tiles(masked):
        a1, a2 = as1_ref[...], as2_ref[...]
        if masked:
            valid = lax.broadcasted_iota(jnp.int32, (bm, bk), 1) < (n - k * bk)
            zero = jnp.zeros((), F32)
            a1 = jnp.where(valid, a1, zero)
            a2 = jnp.where(valid, a2, zero)
        cb = comb_ref[:, ks]
        return _dott(cb, a1), _dott(cb, a2)

    def accumulate(q1, q2):
        @pl.when(k == 0)
        def _():
            acc1_ref[...] = q1
            acc2_ref[...] = q2

        @pl.when(k != 0)
        def _():
            acc1_ref[...] += q1
            acc2_ref[...] += q2

    @pl.when(k != nk - 1)
    def _():
        accumulate(*tiles(masked=False))

    @pl.when(k == nk - 1)
    def _():
        accumulate(*tiles(masked=True))
        r1_ref[...] = jnp.dot(wd1t_ref[...], acc1_ref[...],
                              preferred_element_type=F32)
        r2_ref[...] = jnp.dot(wd2t_ref[...], acc2_ref[...],
                              preferred_element_type=F32)


def _impl(nd, n, d1, d2, o, as1, af1, as2, af2, x1, x2, params,
          we1, we2, wm1at, wm1bt, bm1c, wm2t, bm2c, wd1t, wd2t):
    nloc = n // nd

    bm0 = 2048
    nm0 = -(-n // bm0)
    g1a, g1b, g2a, g2b, c1, c2 = pl.pallas_call(
        functools.partial(_pass0_body, n, bm0),
        grid=(nm0,),
        in_specs=[
            pl.BlockSpec(memory_space=pltpu.SMEM),
            pl.BlockSpec((bm0, d1), lambda i: (i, 0)),
            pl.BlockSpec((bm0, d2), lambda i: (i, 0)),
            pl.BlockSpec((d1, o), lambda i: (0, 0)),
            pl.BlockSpec((d2, o), lambda i: (0, 0)),
        ],
        out_specs=[
            pl.BlockSpec((o, bm0), lambda i: (0, i)),
            pl.BlockSpec((o, bm0), lambda i: (0, i)),
            pl.BlockSpec((o, bm0), lambda i: (0, i)),
            pl.BlockSpec((o, bm0), lambda i: (0, i)),
            pl.BlockSpec((o, 1), lambda i: (0, 0)),
            pl.BlockSpec((o, 1), lambda i: (0, 0)),
        ],
        out_shape=[
            jax.ShapeDtypeStruct((o, n), BF16),
            jax.ShapeDtypeStruct((o, n), BF16),
            jax.ShapeDtypeStruct((o, n), BF16),
            jax.ShapeDtypeStruct((o, n), BF16),
            jax.ShapeDtypeStruct((o, 1), F32),
            jax.ShapeDtypeStruct((o, 1), F32),
        ],
        compiler_params=pltpu.CompilerParams(
            dimension_semantics=("arbitrary",)),
    )(params, x1, x2, we1, we2)

    bm_1 = 512
    bk1 = 2048
    nm1, nk1 = -(-nloc // bm_1), -(-n // bk1)
    kpad1 = nk1 * bk1 - n
    padc = lambda a, p: jnp.pad(a, ((0, 0), (0, p))) if p else a
    adj_spec = pl.BlockSpec((bm_1, bk1), lambda i, k: (i, k))
    thin_spec = pl.BlockSpec((o, nk1 * bk1), lambda i, k: (0, 0))
    col_spec = pl.BlockSpec((o, 1), lambda i, k: (0, 0))
    sq_spec = pl.BlockSpec((o, o), lambda i, k: (0, 0))
    out1_spec = pl.BlockSpec((o, bm_1), lambda i, k: (0, i))
    l1, l2, comb, combb = pl.pallas_call(
        functools.partial(_pass1_body, n, bm_1, bk1),
        grid=(nm1, nk1),
        in_specs=[
            adj_spec, adj_spec, adj_spec, adj_spec,
            thin_spec, thin_spec, thin_spec, thin_spec,
            col_spec, col_spec,
            sq_spec, sq_spec, col_spec, sq_spec, col_spec,
        ],
        out_specs=[out1_spec, out1_spec, out1_spec, out1_spec],
        out_shape=[
            jax.ShapeDtypeStruct((o, nloc), F32),
            jax.ShapeDtypeStruct((o, nloc), F32),
            jax.ShapeDtypeStruct((o, nloc), F32),
            jax.ShapeDtypeStruct((o, nloc), BF16),
        ],
        compiler_params=pltpu.CompilerParams(
            dimension_semantics=("parallel", "arbitrary")),
    )(as1, af1, as2, af2, padc(g1a, kpad1), padc(g1b, kpad1),
      padc(g2a, kpad1), padc(g2b, kpad1), c1, c2,
      wm1at, wm1bt, bm1c, wm2t, bm2c)

    if nd > 1:
        combb = lax.all_gather(combb, 'x', axis=1, tiled=True)

    bm_2 = 1024
    bk2 = 2048
    nm2, nk2 = -(-nloc // bm_2), -(-n // bk2)
    kpad2 = nk2 * bk2 - n
    adj2_spec = pl.BlockSpec((bm_2, bk2), lambda i, k: (i, k))
    r1, r2 = pl.pallas_call(
        functools.partial(_pass2_body, n, bm_2, bk2),
        grid=(nm2, nk2),
        in_specs=[
            adj2_spec, adj2_spec,
            pl.BlockSpec((o, nk2 * bk2), lambda i, k: (0, 0)),
            pl.BlockSpec((d1, o), lambda i, k: (0, 0)),
            pl.BlockSpec((d2, o), lambda i, k: (0, 0)),
        ],
        out_specs=[
            pl.BlockSpec((d1, bm_2), lambda i, k: (0, i)),
            pl.BlockSpec((d2, bm_2), lambda i, k: (0, i)),
        ],
        out_shape=[
            jax.ShapeDtypeStruct((d1, nloc), F32),
            jax.ShapeDtypeStruct((d2, nloc), F32),
        ],
        scratch_shapes=[
            pltpu.VMEM((o, bm_2), F32),
            pltpu.VMEM((o, bm_2), F32),
        ],
        compiler_params=pltpu.CompilerParams(
            dimension_semantics=("parallel", "arbitrary")),
    )(as1, as2, padc(combb, kpad2), wd1t, wd2t)

    return l1, l2, comb, r1, r2


def kernel(features_omics1, features_omics2, adj_spatial_omics1,
           adj_feature_omics1, adj_spatial_omics2, adj_feature_omics2,
           cw1, cb1, cw2, cb2, We1, We2, Wd1, Wd2, Wm1, bm1, Wm2, bm2):
    n, d1 = features_omics1.shape
    d2 = features_omics2.shape[1]
    o = We1.shape[1]

    params = jnp.concatenate([
        cw1.astype(F32), jnp.reshape(cb1, (1,)).astype(F32),
        cw2.astype(F32), jnp.reshape(cb2, (1,)).astype(F32),
    ]).reshape(1, 6)

    devs = jax.devices()
    nd = 2 if (len(devs) >= 2 and n % 2 == 0) else 1
    mesh = Mesh(np.array(devs[:nd]), ('x',))
    shard = P('x', None)
    repl = P(None, None)
    fn = jax.shard_map(
        functools.partial(_impl, nd, n, d1, d2, o),
        mesh=mesh,
        in_specs=(shard, shard, shard, shard) + (repl,) * 12,
        out_specs=(P(None, 'x'),) * 5,
        check_vma=False,
    )
    l1, l2, comb, r1, r2 = fn(
        adj_spatial_omics1, adj_feature_omics1, adj_spatial_omics2,
        adj_feature_omics2, features_omics1, features_omics2, params,
        We1, We2, Wm1[:o].T, Wm1[o:].T, bm1.reshape(o, 1), Wm2.T,
        bm2.reshape(o, 1), Wd1.T, Wd2.T)

    return l1.T, l2.T, comb.T, r1.T, r2.T


# speedup vs baseline: 2.0254x; 2.0254x over previous
import functools

import jax
import jax.numpy as jnp
from jax import lax
from jax.experimental import pallas as pl
from jax.experimental.pallas import tpu as pltpu

F32 = jnp.float32
BF16 = jnp.bfloat16

_DN = (((1,), (1,)), ((), ()))


def _dott(thin, big):
    return lax.dot_general(thin, big.astype(BF16), dimension_numbers=_DN,
                           preferred_element_type=F32,
                           precision=lax.Precision.DEFAULT)


def _pass0_body(n, bm0, params_ref, x1_ref, x2_ref, we1_ref, we2_ref,
                g1a_ref, g1b_ref, g2a_ref, g2b_ref, c1_ref, c2_ref):
    i = pl.program_id(0)
    f1 = lax.dot_general(we1_ref[...], x1_ref[...],
                         dimension_numbers=(((0,), (1,)), ((), ())),
                         preferred_element_type=F32)
    f2 = lax.dot_general(we2_ref[...], x2_ref[...],
                         dimension_numbers=(((0,), (1,)), ((), ())),
                         preferred_element_type=F32)
    g1a_ref[...] = (params_ref[0, 0] * f1).astype(BF16)
    g1b_ref[...] = (params_ref[0, 1] * f1).astype(BF16)
    g2a_ref[...] = (params_ref[0, 3] * f2).astype(BF16)
    g2b_ref[...] = (params_ref[0, 4] * f2).astype(BF16)

    @pl.when(i == 0)
    def _():
        c1_ref[...] = jnp.zeros_like(c1_ref)
        c2_ref[...] = jnp.zeros_like(c2_ref)

    o = f1.shape[0]
    valid = lax.broadcasted_iota(jnp.int32, (o, bm0), 1) < (n - i * bm0)
    zero = jnp.zeros((), F32)
    c1_ref[...] += params_ref[0, 2] * jnp.sum(
        jnp.where(valid, f1, zero), axis=1, keepdims=True)
    c2_ref[...] += params_ref[0, 5] * jnp.sum(
        jnp.where(valid, f2, zero), axis=1, keepdims=True)


def _pass1_body(n, bm, bk, as1_ref, af1_ref, as2_ref, af2_ref,
                g1a_ref, g1b_ref, g2a_ref, g2b_ref, c1_ref, c2_ref,
                wm1at_ref, wm1bt_ref, bm1_ref, wm2t_ref, bm2_ref,
                l1_ref, l2_ref, comb_ref, combb_ref):
    k = pl.program_id(1)
    nk = pl.num_programs(1)
    ks = pl.ds(k * bk, bk)

    def tiles(masked):
        a1, b1 = as1_ref[...], af1_ref[...]
        a2, b2 = as2_ref[...], af2_ref[...]
        if masked:
            valid = lax.broadcasted_iota(jnp.int32, (bm, bk), 1) < (n - k * bk)
            zero = jnp.zeros((), F32)
            a1 = jnp.where(valid, a1, zero)
            b1 = jnp.where(valid, b1, zero)
            a2 = jnp.where(valid, a2, zero)
            b2 = jnp.where(valid, b2, zero)
        p1 = _dott(g1a_ref[:, ks], a1) + _dott(g1b_ref[:, ks], b1)
        p2 = _dott(g2a_ref[:, ks], a2) + _dott(g2b_ref[:, ks], b2)
        return p1, p2

    def accumulate(p1, p2):
        @pl.when(k == 0)
        def _():
            l1_ref[...] = c1_ref[...] + p1
            l2_ref[...] = c2_ref[...] + p2

        @pl.when(k != 0)
        def _():
            l1_ref[...] += p1
            l2_ref[...] += p2

    @pl.when(k != nk - 1)
    def _():
        accumulate(*tiles(masked=False))

    @pl.when(k == nk - 1)
    def _():
        accumulate(*tiles(masked=True))
        l1 = l1_ref[...]
        l2 = l2_ref[...]
        h = (jnp.dot(wm1at_ref[...], l1, preferred_element_type=F32) +
             jnp.dot(wm1bt_ref[...], l2, preferred_element_type=F32) +
             bm1_ref[...])
        comb = jnp.dot(wm2t_ref[...], h,
                       preferred_element_type=F32) + bm2_ref[...]
        comb_ref[...] = comb
        combb_ref[...] = comb.astype(BF16)


def _pass1r_body(n, bm, bk, kj, as1_ref, af1_ref, as2_ref, af2_ref,
                 g1a_ref, g1b_ref, g2a_ref, g2b_ref, c1_ref, c2_ref,
                 wm1at_ref, wm1bt_ref, bm1_ref, wm2t_ref, bm2_ref, combp_ref,
                 l1_ref, l2_ref, comb_ref, combb_ref, r1p_ref, r2p_ref):
    k = pl.program_id(1)
    nk = pl.num_programs(1)
    ks = pl.ds(k * bk, bk)

    def tiles(masked):
        a1, b1 = as1_ref[...], af1_ref[...]
        a2, b2 = as2_ref[...], af2_ref[...]
        if masked:
            valid = lax.broadcasted_iota(jnp.int32, (bm, bk), 1) < (n - k * bk)
            zero = jnp.zeros((), F32)
            a1 = jnp.where(valid, a1, zero)
            b1 = jnp.where(valid, b1, zero)
            a2 = jnp.where(valid, a2, zero)
            b2 = jnp.where(valid, b2, zero)
        p1 = _dott(g1a_ref[:, ks], a1) + _dott(g1b_ref[:, ks], b1)
        p2 = _dott(g2a_ref[:, ks], a2) + _dott(g2b_ref[:, ks], b2)
        return p1, p2

    def accumulate(p1, p2):
        @pl.when(k == 0)
        def _():
            l1_ref[...] = c1_ref[...] + p1
            l2_ref[...] = c2_ref[...] + p2

        @pl.when(k != 0)
        def _():
            l1_ref[...] += p1
            l2_ref[...] += p2

    @pl.when(k != nk - 1)
    def _():
        accumulate(*tiles(masked=False))

    @pl.when(k == nk - 1)
    def _():
        accumulate(*tiles(masked=True))
        l1 = l1_ref[...]
        l2 = l2_ref[...]
        h = (jnp.dot(wm1at_ref[...], l1, preferred_element_type=F32) +
             jnp.dot(wm1bt_ref[...], l2, preferred_element_type=F32) +
             bm1_ref[...])
        comb = jnp.dot(wm2t_ref[...], h,
                       preferred_element_type=F32) + bm2_ref[...]
        comb_ref[...] = comb
        combb_ref[...] = comb.astype(BF16)

    @pl.when(k == 0)
    def _():
        r1p_ref[...] = _dott(combp_ref[:, ks], as1_ref[...])
        r2p_ref[...] = _dott(combp_ref[:, ks], as2_ref[...])

    @pl.when(jnp.logical_and(k > 0, k < kj))
    def _():
        r1p_ref[...] += _dott(combp_ref[:, ks], as1_ref[...])
        r2p_ref[...] += _dott(combp_ref[:, ks], as2_ref[...])


def _pass2_body(n, bm, bk, kj, as1_ref, as2_ref, comb_ref, wd1t_ref,
                wd2t_ref, r1p_ref, r2p_ref, r1_ref, r2_ref,
                acc1_ref, acc2_ref):
    k = pl.program_id(1)
    nk = pl.num_programs(1)
    ks = pl.ds((k + kj) * bk, bk)

    def tiles(masked):
        a1, a2 = as1_ref[...], as2_ref[...]
        if masked:
            valid = lax.broadcasted_iota(jnp.int32, (bm, bk), 1) \
                < (n - (k + kj) * bk)
            zero = jnp.zeros((), F32)
            a1 = jnp.where(valid, a1, zero)
            a2 = jnp.where(valid, a2, zero)
        cb = comb_ref[:, ks]
        return _dott(cb, a1), _dott(cb, a2)

    def accumulate(q1, q2):
        @pl.when(k == 0)
        def _():
            acc1_ref[...] = r1p_ref[...] + q1
            acc2_ref[...] = r2p_ref[...] + q2

        @pl.when(k != 0)
        def _():
            acc1_ref[...] += q1
            acc2_ref[...] += q2

    @pl.when(k != nk - 1)
    def _():
        accumulate(*tiles(masked=False))

    @pl.when(k == nk - 1)
    def _():
        accumulate(*tiles(masked=True))
        r1_ref[...] = jnp.dot(wd1t_ref[...], acc1_ref[...],
                              preferred_element_type=F32)
        r2_ref[...] = jnp.dot(wd2t_ref[...], acc2_ref[...],
                              preferred_element_type=F32)


def kernel(features_omics1, features_omics2, adj_spatial_omics1,
           adj_feature_omics1, adj_spatial_omics2, adj_feature_omics2,
           cw1, cb1, cw2, cb2, We1, We2, Wd1, Wd2, Wm1, bm1, Wm2, bm2):
    n, d1 = features_omics1.shape
    d2 = features_omics2.shape[1]
    o = We1.shape[1]

    params = jnp.concatenate([
        cw1.astype(F32), jnp.reshape(cb1, (1,)).astype(F32),
        cw2.astype(F32), jnp.reshape(cb2, (1,)).astype(F32),
    ]).reshape(1, 6)

    bm0 = 2048
    nm0 = -(-n // bm0)
    g1a, g1b, g2a, g2b, c1, c2 = pl.pallas_call(
        functools.partial(_pass0_body, n, bm0),
        grid=(nm0,),
        in_specs=[
            pl.BlockSpec(memory_space=pltpu.SMEM),
            pl.BlockSpec((bm0, d1), lambda i: (i, 0)),
            pl.BlockSpec((bm0, d2), lambda i: (i, 0)),
            pl.BlockSpec((d1, o), lambda i: (0, 0)),
            pl.BlockSpec((d2, o), lambda i: (0, 0)),
        ],
        out_specs=[
            pl.BlockSpec((o, bm0), lambda i: (0, i)),
            pl.BlockSpec((o, bm0), lambda i: (0, i)),
            pl.BlockSpec((o, bm0), lambda i: (0, i)),
            pl.BlockSpec((o, bm0), lambda i: (0, i)),
            pl.BlockSpec((o, 1), lambda i: (0, 0)),
            pl.BlockSpec((o, 1), lambda i: (0, 0)),
        ],
        out_shape=[
            jax.ShapeDtypeStruct((o, n), BF16),
            jax.ShapeDtypeStruct((o, n), BF16),
            jax.ShapeDtypeStruct((o, n), BF16),
            jax.ShapeDtypeStruct((o, n), BF16),
            jax.ShapeDtypeStruct((o, 1), F32),
            jax.ShapeDtypeStruct((o, 1), F32),
        ],
        compiler_params=pltpu.CompilerParams(
            dimension_semantics=("arbitrary",)),
    )(params, features_omics1, features_omics2, We1, We2)

    bm = 1024
    bk = 1024
    nk = -(-n // bk)
    kpad = nk * bk - n
    padc = lambda a, p: jnp.pad(a, ((0, 0), (0, p))) if p else a
    g1a, g1b, g2a, g2b = (padc(g, kpad) for g in (g1a, g1b, g2a, g2b))

    nchunks = 4
    step = max(1, round(n / nchunks / bm)) * bm
    edges = sorted({min(j * step, n) for j in range(nchunks)} | {n})
    chunks = [(lo, hi) for lo, hi in zip(edges[:-1], edges[1:]) if hi > lo]

    thin_spec = pl.BlockSpec((o, nk * bk), lambda i, k: (0, 0))
    col_spec = pl.BlockSpec((o, 1), lambda i, k: (0, 0))
    sq_spec = pl.BlockSpec((o, o), lambda i, k: (0, 0))
    wm_args = (Wm1[:o].T, Wm1[o:].T, bm1.reshape(o, 1), Wm2.T,
               bm2.reshape(o, 1))

    l1c, l2c, combc, combbc, r1pc, r2pc = [], [], [], [], [], []
    for lo, hi in chunks:
        rb, kj, w = lo // bm, lo // bk, hi - lo
        nmj = -(-w // bm)
        adj_spec = pl.BlockSpec((bm, bk), lambda i, k, rb=rb: (i + rb, k))
        outw_spec = pl.BlockSpec((o, bm), lambda i, k: (0, i))
        out_shapes = [
            jax.ShapeDtypeStruct((o, w), F32),
            jax.ShapeDtypeStruct((o, w), F32),
            jax.ShapeDtypeStruct((o, w), F32),
            jax.ShapeDtypeStruct((o, w), BF16),
        ]
        in_specs = [adj_spec] * 4 + [thin_spec] * 4 + [col_spec] * 2 + [
            sq_spec, sq_spec, col_spec, sq_spec, col_spec]
        args = (adj_spatial_omics1, adj_feature_omics1, adj_spatial_omics2,
                adj_feature_omics2, g1a, g1b, g2a, g2b, c1, c2) + wm_args
        if kj == 0:
            body = functools.partial(_pass1_body, n, bm, bk)
        else:
            combp = jnp.concatenate(combbc, axis=1)
            body = functools.partial(_pass1r_body, n, bm, bk, kj)
            in_specs = in_specs + [
                pl.BlockSpec((o, kj * bk), lambda i, k: (0, 0))]
            args = args + (combp,)
            out_shapes = out_shapes + [
                jax.ShapeDtypeStruct((o, w), F32),
                jax.ShapeDtypeStruct((o, w), F32),
            ]
        outs = pl.pallas_call(
            body,
            grid=(nmj, nk),
            in_specs=in_specs,
            out_specs=[outw_spec] * len(out_shapes),
            out_shape=out_shapes,
            compiler_params=pltpu.CompilerParams(
                dimension_semantics=("parallel", "arbitrary")),
        )(*args)
        l1c.append(outs[0])
        l2c.append(outs[1])
        combc.append(outs[2])
        combbc.append(outs[3])
        r1pc.append(outs[4] if kj else jnp.zeros((o, w), F32))
        r2pc.append(outs[5] if kj else jnp.zeros((o, w), F32))

    combb = padc(jnp.concatenate(combbc, axis=1), kpad)
    r1c, r2c = [], []
    for idx, (lo, hi) in enumerate(chunks):
        rb, kj, w = lo // bm, lo // bk, hi - lo
        nmj = -(-w // bm)
        adj_spec = pl.BlockSpec(
            (bm, bk), lambda i, k, rb=rb, kj=kj: (i + rb, k + kj))
        r1, r2 = pl.pallas_call(
            functools.partial(_pass2_body, n, bm, bk, kj),
            grid=(nmj, nk - kj),
            in_specs=[
                adj_spec, adj_spec,
                pl.BlockSpec((o, nk * bk), lambda i, k: (0, 0)),
                pl.BlockSpec((d1, o), lambda i, k: (0, 0)),
                pl.BlockSpec((d2, o), lambda i, k: (0, 0)),
                pl.BlockSpec((o, bm), lambda i, k: (0, i)),
                pl.BlockSpec((o, bm), lambda i, k: (0, i)),
            ],
            out_specs=[
                pl.BlockSpec((d1, bm), lambda i, k: (0, i)),
                pl.BlockSpec((d2, bm), lambda i, k: (0, i)),
            ],
            out_shape=[
                jax.ShapeDtypeStruct((d1, w), F32),
                jax.ShapeDtypeStruct((d2, w), F32),
            ],
            scratch_shapes=[
                pltpu.VMEM((o, bm), F32),
                pltpu.VMEM((o, bm), F32),
            ],
            compiler_params=pltpu.CompilerParams(
                dimension_semantics=("parallel", "arbitrary")),
        )(adj_spatial_omics1, adj_spatial_omics2, combb, Wd1.T, Wd2.T,
          r1pc[idx], r2pc[idx])
        r1c.append(r1)
        r2c.append(r2)

    cat = lambda xs: jnp.concatenate(xs, axis=1)
    return (cat(l1c).T, cat(l2c).T, cat(combc).T, cat(r1c).T, cat(r2c).T)


# speedup vs baseline: 2.0412x; 1.0078x over previous
import functools

import jax
import jax.numpy as jnp
from jax import lax
from jax.experimental import pallas as pl
from jax.experimental.pallas import tpu as pltpu

F32 = jnp.float32
BF16 = jnp.bfloat16

_DN = (((1,), (1,)), ((), ()))


def _dott(thin, big):
    return lax.dot_general(thin, big.astype(BF16), dimension_numbers=_DN,
                           preferred_element_type=F32,
                           precision=lax.Precision.DEFAULT)


def _pass0_body(n, bm0, params_ref, x1_ref, x2_ref, we1_ref, we2_ref,
                g1a_ref, g1b_ref, g2a_ref, g2b_ref, c1_ref, c2_ref):
    i = pl.program_id(0)
    f1 = lax.dot_general(we1_ref[...], x1_ref[...],
                         dimension_numbers=(((0,), (1,)), ((), ())),
                         preferred_element_type=F32)
    f2 = lax.dot_general(we2_ref[...], x2_ref[...],
                         dimension_numbers=(((0,), (1,)), ((), ())),
                         preferred_element_type=F32)
    g1a_ref[...] = (params_ref[0, 0] * f1).astype(BF16)
    g1b_ref[...] = (params_ref[0, 1] * f1).astype(BF16)
    g2a_ref[...] = (params_ref[0, 3] * f2).astype(BF16)
    g2b_ref[...] = (params_ref[0, 4] * f2).astype(BF16)

    @pl.when(i == 0)
    def _():
        c1_ref[...] = jnp.zeros_like(c1_ref)
        c2_ref[...] = jnp.zeros_like(c2_ref)

    o = f1.shape[0]
    valid = lax.broadcasted_iota(jnp.int32, (o, bm0), 1) < (n - i * bm0)
    zero = jnp.zeros((), F32)
    c1_ref[...] += params_ref[0, 2] * jnp.sum(
        jnp.where(valid, f1, zero), axis=1, keepdims=True)
    c2_ref[...] += params_ref[0, 5] * jnp.sum(
        jnp.where(valid, f2, zero), axis=1, keepdims=True)


def _pass1_body(n, bm, bk, as1_ref, af1_ref, as2_ref, af2_ref,
                g1a_ref, g1b_ref, g2a_ref, g2b_ref, c1_ref, c2_ref,
                wm1at_ref, wm1bt_ref, bm1_ref, wm2t_ref, bm2_ref,
                l1_ref, l2_ref, comb_ref, combb_ref):
    k = pl.program_id(1)
    nk = pl.num_programs(1)
    ks = pl.ds(k * bk, bk)

    def tiles(masked):
        a1, b1 = as1_ref[...], af1_ref[...]
        a2, b2 = as2_ref[...], af2_ref[...]
        if masked:
            valid = lax.broadcasted_iota(jnp.int32, (bm, bk), 1) < (n - k * bk)
            zero = jnp.zeros((), F32)
            a1 = jnp.where(valid, a1, zero)
            b1 = jnp.where(valid, b1, zero)
            a2 = jnp.where(valid, a2, zero)
            b2 = jnp.where(valid, b2, zero)
        p1 = _dott(g1a_ref[:, ks], a1) + _dott(g1b_ref[:, ks], b1)
        p2 = _dott(g2a_ref[:, ks], a2) + _dott(g2b_ref[:, ks], b2)
        return p1, p2

    def accumulate(p1, p2):
        @pl.when(k == 0)
        def _():
            l1_ref[...] = c1_ref[...] + p1
            l2_ref[...] = c2_ref[...] + p2

        @pl.when(k != 0)
        def _():
            l1_ref[...] += p1
            l2_ref[...] += p2

    @pl.when(k != nk - 1)
    def _():
        accumulate(*tiles(masked=False))

    @pl.when(k == nk - 1)
    def _():
        accumulate(*tiles(masked=True))
        l1 = l1_ref[...]
        l2 = l2_ref[...]
        h = (jnp.dot(wm1at_ref[...], l1, preferred_element_type=F32) +
             jnp.dot(wm1bt_ref[...], l2, preferred_element_type=F32) +
             bm1_ref[...])
        comb = jnp.dot(wm2t_ref[...], h,
                       preferred_element_type=F32) + bm2_ref[...]
        comb_ref[...] = comb
        combb_ref[...] = comb.astype(BF16)


def _pass1r_body(n, bm, bk, kj, as1_ref, af1_ref, as2_ref, af2_ref,
                 g1a_ref, g1b_ref, g2a_ref, g2b_ref, c1_ref, c2_ref,
                 wm1at_ref, wm1bt_ref, bm1_ref, wm2t_ref, bm2_ref, combp_ref,
                 l1_ref, l2_ref, comb_ref, combb_ref, r1p_ref, r2p_ref):
    k = pl.program_id(1)
    nk = pl.num_programs(1)
    ks = pl.ds(k * bk, bk)
    o = g1a_ref.shape[0]

    def tiles(masked):
        a1, b1 = as1_ref[...], af1_ref[...]
        a2, b2 = as2_ref[...], af2_ref[...]
        if masked:
            valid = lax.broadcasted_iota(jnp.int32, (bm, bk), 1) < (n - k * bk)
            zero = jnp.zeros((), F32)
            a1 = jnp.where(valid, a1, zero)
            b1 = jnp.where(valid, b1, zero)
            a2 = jnp.where(valid, a2, zero)
            b2 = jnp.where(valid, b2, zero)
        p1 = _dott(g1a_ref[:, ks], a1) + _dott(g1b_ref[:, ks], b1)
        p2 = _dott(g2a_ref[:, ks], a2) + _dott(g2b_ref[:, ks], b2)
        return p1, p2

    def accumulate(p1, p2):
        @pl.when(k == 0)
        def _():
            l1_ref[...] = c1_ref[...] + p1
            l2_ref[...] = c2_ref[...] + p2

        @pl.when(k != 0)
        def _():
            l1_ref[...] += p1
            l2_ref[...] += p2

    @pl.when(k < kj)
    def _():
        cp = combp_ref[:, ks]
        t1 = _dott(jnp.concatenate([g1a_ref[:, ks], cp], axis=0),
                   as1_ref[...])
        t2 = _dott(jnp.concatenate([g2a_ref[:, ks], cp], axis=0),
                   as2_ref[...])
        p1 = t1[:o] + _dott(g1b_ref[:, ks], af1_ref[...])
        p2 = t2[:o] + _dott(g2b_ref[:, ks], af2_ref[...])
        accumulate(p1, p2)

        @pl.when(k == 0)
        def _():
            r1p_ref[...] = t1[o:]
            r2p_ref[...] = t2[o:]

        @pl.when(k != 0)
        def _():
            r1p_ref[...] += t1[o:]
            r2p_ref[...] += t2[o:]

    @pl.when(jnp.logical_and(k >= kj, k != nk - 1))
    def _():
        accumulate(*tiles(masked=False))

    @pl.when(k == nk - 1)
    def _():
        accumulate(*tiles(masked=True))
        l1 = l1_ref[...]
        l2 = l2_ref[...]
        h = (jnp.dot(wm1at_ref[...], l1, preferred_element_type=F32) +
             jnp.dot(wm1bt_ref[...], l2, preferred_element_type=F32) +
             bm1_ref[...])
        comb = jnp.dot(wm2t_ref[...], h,
                       preferred_element_type=F32) + bm2_ref[...]
        comb_ref[...] = comb
        combb_ref[...] = comb.astype(BF16)


def _pass2_body(n, bm, bk, kj, as1_ref, as2_ref, comb_ref, wd1t_ref,
                wd2t_ref, r1p_ref, r2p_ref, r1_ref, r2_ref,
                acc1_ref, acc2_ref):
    k = pl.program_id(1)
    nk = pl.num_programs(1)
    ks = pl.ds((k + kj) * bk, bk)

    def tiles(masked):
        a1, a2 = as1_ref[...], as2_ref[...]
        if masked:
            valid = lax.broadcasted_iota(jnp.int32, (bm, bk), 1) \
                < (n - (k + kj) * bk)
            zero = jnp.zeros((), F32)
            a1 = jnp.where(valid, a1, zero)
            a2 = jnp.where(valid, a2, zero)
        cb = comb_ref[:, ks]
        return _dott(cb, a1), _dott(cb, a2)

    def accumulate(q1, q2):
        @pl.when(k == 0)
        def _():
            acc1_ref[...] = r1p_ref[...] + q1
            acc2_ref[...] = r2p_ref[...] + q2

        @pl.when(k != 0)
        def _():
            acc1_ref[...] += q1
            acc2_ref[...] += q2

    @pl.when(k != nk - 1)
    def _():
        accumulate(*tiles(masked=False))

    @pl.when(k == nk - 1)
    def _():
        accumulate(*tiles(masked=True))
        r1_ref[...] = jnp.dot(wd1t_ref[...], acc1_ref[...],
                              preferred_element_type=F32)
        r2_ref[...] = jnp.dot(wd2t_ref[...], acc2_ref[...],
                              preferred_element_type=F32)


def kernel(features_omics1, features_omics2, adj_spatial_omics1,
           adj_feature_omics1, adj_spatial_omics2, adj_feature_omics2,
           cw1, cb1, cw2, cb2, We1, We2, Wd1, Wd2, Wm1, bm1, Wm2, bm2):
    n, d1 = features_omics1.shape
    d2 = features_omics2.shape[1]
    o = We1.shape[1]

    params = jnp.concatenate([
        cw1.astype(F32), jnp.reshape(cb1, (1,)).astype(F32),
        cw2.astype(F32), jnp.reshape(cb2, (1,)).astype(F32),
    ]).reshape(1, 6)

    bm0 = 2048
    nm0 = -(-n // bm0)
    g1a, g1b, g2a, g2b, c1, c2 = pl.pallas_call(
        functools.partial(_pass0_body, n, bm0),
        grid=(nm0,),
        in_specs=[
            pl.BlockSpec(memory_space=pltpu.SMEM),
            pl.BlockSpec((bm0, d1), lambda i: (i, 0)),
            pl.BlockSpec((bm0, d2), lambda i: (i, 0)),
            pl.BlockSpec((d1, o), lambda i: (0, 0)),
            pl.BlockSpec((d2, o), lambda i: (0, 0)),
        ],
        out_specs=[
            pl.BlockSpec((o, bm0), lambda i: (0, i)),
            pl.BlockSpec((o, bm0), lambda i: (0, i)),
            pl.BlockSpec((o, bm0), lambda i: (0, i)),
            pl.BlockSpec((o, bm0), lambda i: (0, i)),
            pl.BlockSpec((o, 1), lambda i: (0, 0)),
            pl.BlockSpec((o, 1), lambda i: (0, 0)),
        ],
        out_shape=[
            jax.ShapeDtypeStruct((o, n), BF16),
            jax.ShapeDtypeStruct((o, n), BF16),
            jax.ShapeDtypeStruct((o, n), BF16),
            jax.ShapeDtypeStruct((o, n), BF16),
            jax.ShapeDtypeStruct((o, 1), F32),
            jax.ShapeDtypeStruct((o, 1), F32),
        ],
        compiler_params=pltpu.CompilerParams(
            dimension_semantics=("arbitrary",)),
    )(params, features_omics1, features_omics2, We1, We2)

    bm = 1024
    bk = 1024
    nk = -(-n // bk)
    kpad = nk * bk - n
    padc = lambda a, p: jnp.pad(a, ((0, 0), (0, p))) if p else a
    g1a, g1b, g2a, g2b = (padc(g, kpad) for g in (g1a, g1b, g2a, g2b))

    nchunks = 4
    step = max(1, round(n / nchunks / bm)) * bm
    edges = sorted({min(j * step, n) for j in range(nchunks)} | {n})
    chunks = [(lo, hi) for lo, hi in zip(edges[:-1], edges[1:]) if hi > lo]

    thin_spec = pl.BlockSpec((o, nk * bk), lambda i, k: (0, 0))
    col_spec = pl.BlockSpec((o, 1), lambda i, k: (0, 0))
    sq_spec = pl.BlockSpec((o, o), lambda i, k: (0, 0))
    wm_args = (Wm1[:o].T, Wm1[o:].T, bm1.reshape(o, 1), Wm2.T,
               bm2.reshape(o, 1))

    l1c, l2c, combc, combbc, r1pc, r2pc = [], [], [], [], [], []
    for lo, hi in chunks:
        rb, kj, w = lo // bm, lo // bk, hi - lo
        nmj = -(-w // bm)
        adj_spec = pl.BlockSpec((bm, bk), lambda i, k, rb=rb: (i + rb, k))
        outw_spec = pl.BlockSpec((o, bm), lambda i, k: (0, i))
        out_shapes = [
            jax.ShapeDtypeStruct((o, w), F32),
            jax.ShapeDtypeStruct((o, w), F32),
            jax.ShapeDtypeStruct((o, w), F32),
            jax.ShapeDtypeStruct((o, w), BF16),
        ]
        in_specs = [adj_spec] * 4 + [thin_spec] * 4 + [col_spec] * 2 + [
            sq_spec, sq_spec, col_spec, sq_spec, col_spec]
        args = (adj_spatial_omics1, adj_feature_omics1, adj_spatial_omics2,
                adj_feature_omics2, g1a, g1b, g2a, g2b, c1, c2) + wm_args
        if kj == 0:
            body = functools.partial(_pass1_body, n, bm, bk)
        else:
            combp = jnp.concatenate(combbc, axis=1)
            body = functools.partial(_pass1r_body, n, bm, bk, kj)
            in_specs = in_specs + [
                pl.BlockSpec((o, kj * bk), lambda i, k: (0, 0))]
            args = args + (combp,)
            out_shapes = out_shapes + [
                jax.ShapeDtypeStruct((o, w), F32),
                jax.ShapeDtypeStruct((o, w), F32),
            ]
        outs = pl.pallas_call(
            body,
            grid=(nmj, nk),
            in_specs=in_specs,
            out_specs=[outw_spec] * len(out_shapes),
            out_shape=out_shapes,
            compiler_params=pltpu.CompilerParams(
                dimension_semantics=("parallel", "arbitrary")),
        )(*args)
        l1c.append(outs[0])
        l2c.append(outs[1])
        combc.append(outs[2])
        combbc.append(outs[3])
        r1pc.append(outs[4] if kj else jnp.zeros((o, w), F32))
        r2pc.append(outs[5] if kj else jnp.zeros((o, w), F32))

    combb = padc(jnp.concatenate(combbc, axis=1), kpad)
    r1c, r2c = [], []
    for idx, (lo, hi) in enumerate(chunks):
        rb, kj, w = lo // bm, lo // bk, hi - lo
        nmj = -(-w // bm)
        adj_spec = pl.BlockSpec(
            (bm, bk), lambda i, k, rb=rb, kj=kj: (i + rb, k + kj))
        r1, r2 = pl.pallas_call(
            functools.partial(_pass2_body, n, bm, bk, kj),
            grid=(nmj, nk - kj),
            in_specs=[
                adj_spec, adj_spec,
                pl.BlockSpec((o, nk * bk), lambda i, k: (0, 0)),
                pl.BlockSpec((d1, o), lambda i, k: (0, 0)),
                pl.BlockSpec((d2, o), lambda i, k: (0, 0)),
                pl.BlockSpec((o, bm), lambda i, k: (0, i)),
                pl.BlockSpec((o, bm), lambda i, k: (0, i)),
            ],
            out_specs=[
                pl.BlockSpec((d1, bm), lambda i, k: (0, i)),
                pl.BlockSpec((d2, bm), lambda i, k: (0, i)),
            ],
            out_shape=[
                jax.ShapeDtypeStruct((d1, w), F32),
                jax.ShapeDtypeStruct((d2, w), F32),
            ],
            scratch_shapes=[
                pltpu.VMEM((o, bm), F32),
                pltpu.VMEM((o, bm), F32),
            ],
            compiler_params=pltpu.CompilerParams(
                dimension_semantics=("parallel", "arbitrary")),
        )(adj_spatial_omics1, adj_spatial_omics2, combb, Wd1.T, Wd2.T,
          r1pc[idx], r2pc[idx])
        r1c.append(r1)
        r2c.append(r2)

    cat = lambda xs: jnp.concatenate(xs, axis=1)
    return (cat(l1c).T, cat(l2c).T, cat(combc).T, cat(r1c).T, cat(r2c).T)
